# baseline (speedup 1.0000x reference)
import jax
import jax.numpy as jnp
from jax.experimental import pallas as pl

BLOCK_S = 2048


def _add_body(x_ref, e_ref, o_ref):
    o_ref[0] = x_ref[0] + e_ref[...]


def kernel(inputs, embeddings):
    b, s, d = inputs.shape
    emb = embeddings[:s]
    grid = (b, s // BLOCK_S)
    return pl.pallas_call(
        _add_body,
        grid=grid,
        in_specs=[
            pl.BlockSpec((1, BLOCK_S, d), lambda i, j: (i, j, 0)),
            pl.BlockSpec((BLOCK_S, d), lambda i, j: (j, 0)),
        ],
        out_specs=pl.BlockSpec((1, BLOCK_S, d), lambda i, j: (i, j, 0)),
        out_shape=jax.ShapeDtypeStruct((b, s, d), inputs.dtype),
    )(inputs, emb)


# speedup vs baseline: 1.3307x; 1.3307x over previous
import jax
import jax.numpy as jnp
from jax import lax
from jax.experimental import pallas as pl
from jax.experimental.pallas import tpu as pltpu

CS = 512
NBUF = 4


def _body(x_hbm, e_hbm, o_hbm, in_buf, out_buf, emb_buf, in_sem, out_sem,
          emb_sem):
    b, s, d = x_hbm.shape
    n_s = s // CS
    total = b * n_s

    def in_copy(t, slot):
        bi = t // n_s
        si = lax.rem(t, n_s)
        return pltpu.make_async_copy(
            x_hbm.at[bi, pl.ds(si * CS, CS), :], in_buf.at[slot],
            in_sem.at[slot])

    def out_copy(t, slot):
        bi = t // n_s
        si = lax.rem(t, n_s)
        return pltpu.make_async_copy(
            out_buf.at[slot], o_hbm.at[bi, pl.ds(si * CS, CS), :],
            out_sem.at[slot])

    for c in range(n_s):
        pltpu.make_async_copy(
            e_hbm.at[pl.ds(c * CS, CS), :],
            emb_buf.at[pl.ds(c * CS, CS), :], emb_sem.at[c]).start()
    for k in range(NBUF):
        in_copy(k, k).start()

    def step(t, carry):
        slot = lax.rem(t, NBUF)
        si = lax.rem(t, n_s)
        in_copy(t, slot).wait()

        @pl.when(t < n_s)
        def _():
            pltpu.make_async_copy(
                e_hbm.at[pl.ds(0, CS), :], emb_buf.at[pl.ds(0, CS), :],
                emb_sem.at[si]).wait()

        @pl.when(t >= NBUF)
        def _():
            out_copy(t - NBUF, slot).wait()

        out_buf[slot] = in_buf[slot] + emb_buf[pl.ds(si * CS, CS), :]
        out_copy(t, slot).start()

        @pl.when(t + NBUF < total)
        def _():
            in_copy(t + NBUF, slot).start()

        return carry

    lax.fori_loop(0, total, step, 0)

    for k in range(NBUF):
        slot = (total - NBUF + k) % NBUF
        out_copy(total - NBUF + k, slot).wait()


def kernel(inputs, embeddings):
    b, s, d = inputs.shape
    emb = embeddings[:s]
    return pl.pallas_call(
        _body,
        in_specs=[
            pl.BlockSpec(memory_space=pl.ANY),
            pl.BlockSpec(memory_space=pl.ANY),
        ],
        out_specs=pl.BlockSpec(memory_space=pl.ANY),
        out_shape=jax.ShapeDtypeStruct((b, s, d), inputs.dtype),
        scratch_shapes=[
            pltpu.VMEM((NBUF, CS, d), jnp.float32),
            pltpu.VMEM((NBUF, CS, d), jnp.float32),
            pltpu.VMEM((s, d), jnp.float32),
            pltpu.SemaphoreType.DMA((NBUF,)),
            pltpu.SemaphoreType.DMA((NBUF,)),
            pltpu.SemaphoreType.DMA((s // CS,)),
        ],
    )(inputs, emb)
